# baseline (speedup 1.0000x reference)
import jax
import jax.numpy as jnp
from jax.experimental import pallas as pl
from jax.experimental.pallas import tpu as pltpu


_D = 262144
_N = 32
_K1 = 15
_B1 = 8192
_NB1 = _D // _B1
_B2 = 8192
_NB2 = _D // _B2


def _selection_vector(g):
    rio = jax.lax.broadcasted_iota(jnp.int32, (_N, _N), 0)
    cio = jax.lax.broadcasted_iota(jnp.int32, (_N, _N), 1)
    eye = rio == cio
    zero = jnp.zeros_like(g)
    sq_col = jnp.sum(jnp.where(eye, g, zero), axis=1, keepdims=True)
    sq_row = jnp.sum(jnp.where(eye, g, zero), axis=0, keepdims=True)
    d2 = sq_col + sq_row - 2.0 * g
    cd = jnp.sqrt(jnp.maximum(d2, 0.0))

    vals = cd
    rowmask = jnp.zeros((_N, _N), jnp.float32)
    s15 = jnp.zeros((_N, 1), jnp.float32)
    big = jnp.float32(3.0e38)
    for _ in range(_K1):
        m = jnp.min(vals, axis=1, keepdims=True)
        s15 = s15 + m
        is_min = vals <= m
        fidx = jnp.min(jnp.where(is_min, cio, _N), axis=1, keepdims=True)
        sel = cio == fidx
        rowmask = jnp.where(sel, 1.0, rowmask)
        vals = jnp.where(sel, big, vals)

    mn = jnp.min(s15)
    rio1 = jax.lax.broadcasted_iota(jnp.int32, (_N, 1), 0)
    istar = jnp.min(jnp.where(s15 <= mn, rio1, _N))

    msel = jnp.where(rio == istar, rowmask, zero)
    ones = jnp.ones((_N, 1), jnp.float32)
    mcol = jax.lax.dot_general(
        msel, ones, (((0,), (0,)), ((), ())), preferred_element_type=jnp.float32
    )
    return mcol * (1.0 / _K1)


def _mkrum_kernel(x_ref, out_ref, gacc_ref, mcol_ref):
    i = pl.program_id(0)

    @pl.when(i < _NB1)
    def _phase1():
        xblk = x_ref[...]
        part = jax.lax.dot_general(
            xblk, xblk, (((0,), (0,)), ((), ())),
            preferred_element_type=jnp.float32,
        )

        @pl.when(i == 0)
        def _():
            gacc_ref[...] = part

        @pl.when(i > 0)
        def _():
            gacc_ref[...] = gacc_ref[...] + part

    @pl.when(i == _NB1)
    def _boundary():
        mcol_ref[...] = _selection_vector(gacc_ref[...])

    @pl.when(i >= _NB1)
    def _phase2():
        out_ref[...] = jax.lax.dot_general(
            x_ref[...], mcol_ref[...], (((1,), (0,)), ((), ())),
            preferred_element_type=jnp.float32,
        )


@jax.jit
def kernel(input):
    x = jnp.reshape(input, (_D, _N))

    def in_map(i):
        return (jnp.where(i < _NB1, i, i - _NB1), 0)

    out = pl.pallas_call(
        _mkrum_kernel,
        grid=(_NB1 + _NB2,),
        in_specs=[pl.BlockSpec((_B1, _N), in_map)],
        out_specs=pl.BlockSpec((_B2, 1), lambda i: (jnp.maximum(i - _NB1, 0), 0)),
        out_shape=jax.ShapeDtypeStruct((_D, 1), jnp.float32),
        scratch_shapes=[
            pltpu.VMEM((_N, _N), jnp.float32),
            pltpu.VMEM((_N, 1), jnp.float32),
        ],
    )(x)
    return jnp.reshape(out, (1, _D, 1))


# speedup vs baseline: 1.0594x; 1.0594x over previous
import jax
import jax.numpy as jnp
from jax.experimental import pallas as pl
from jax.experimental.pallas import tpu as pltpu


_D = 262144
_N = 32
_K1 = 15
_B1 = 16384
_NB1 = _D // _B1
_B2 = 16384
_NB2 = _D // _B2


def _selection_vector(g):
    rio = jax.lax.broadcasted_iota(jnp.int32, (_N, _N), 0)
    cio = jax.lax.broadcasted_iota(jnp.int32, (_N, _N), 1)
    eye = rio == cio
    zero = jnp.zeros_like(g)
    sq_col = jnp.sum(jnp.where(eye, g, zero), axis=1, keepdims=True)
    sq_row = jnp.sum(jnp.where(eye, g, zero), axis=0, keepdims=True)
    d2 = sq_col + sq_row - 2.0 * g
    cd = jnp.sqrt(jnp.maximum(d2, 0.0))

    vals = cd
    rowmask = jnp.zeros((_N, _N), jnp.float32)
    s15 = jnp.zeros((_N, 1), jnp.float32)
    big = jnp.float32(3.0e38)
    for _ in range(_K1):
        m = jnp.min(vals, axis=1, keepdims=True)
        s15 = s15 + m
        is_min = vals <= m
        fidx = jnp.min(jnp.where(is_min, cio, _N), axis=1, keepdims=True)
        sel = cio == fidx
        rowmask = jnp.where(sel, 1.0, rowmask)
        vals = jnp.where(sel, big, vals)

    mn = jnp.min(s15)
    rio1 = jax.lax.broadcasted_iota(jnp.int32, (_N, 1), 0)
    istar = jnp.min(jnp.where(s15 <= mn, rio1, _N))

    msel = jnp.where(rio == istar, rowmask, zero)
    ones = jnp.ones((_N, 1), jnp.float32)
    mcol = jax.lax.dot_general(
        msel, ones, (((0,), (0,)), ((), ())), preferred_element_type=jnp.float32
    )
    return mcol * (1.0 / _K1)


def _mkrum_kernel(x_ref, out_ref, gacc_ref, mcol_ref):
    i = pl.program_id(0)

    @pl.when(i < _NB1)
    def _phase1():
        xblk = x_ref[...]
        part = jax.lax.dot_general(
            xblk, xblk, (((0,), (0,)), ((), ())),
            preferred_element_type=jnp.float32,
        )

        @pl.when(i == 0)
        def _():
            gacc_ref[...] = part

        @pl.when(i > 0)
        def _():
            gacc_ref[...] = gacc_ref[...] + part

    @pl.when(i == _NB1)
    def _boundary():
        mcol_ref[...] = _selection_vector(gacc_ref[...])

    @pl.when(i >= _NB1)
    def _phase2():
        out_ref[...] = jax.lax.dot_general(
            x_ref[...], mcol_ref[...], (((1,), (0,)), ((), ())),
            preferred_element_type=jnp.float32,
        )


@jax.jit
def kernel(input):
    x = jnp.reshape(input, (_D, _N))

    def in_map(i):
        return (jnp.where(i < _NB1, i, i - _NB1), 0)

    out = pl.pallas_call(
        _mkrum_kernel,
        grid=(_NB1 + _NB2,),
        in_specs=[pl.BlockSpec((_B1, _N), in_map)],
        out_specs=pl.BlockSpec((_B2, 1), lambda i: (jnp.maximum(i - _NB1, 0), 0)),
        out_shape=jax.ShapeDtypeStruct((_D, 1), jnp.float32),
        scratch_shapes=[
            pltpu.VMEM((_N, _N), jnp.float32),
            pltpu.VMEM((_N, 1), jnp.float32),
        ],
    )(x)
    return jnp.reshape(out, (1, _D, 1))


# speedup vs baseline: 5.9260x; 5.5938x over previous
import jax
import jax.numpy as jnp
from jax.experimental import pallas as pl
from jax.experimental.pallas import tpu as pltpu


_D = 262144
_N = 32
_K1 = 15
_C = 16384
_NB = _D // _C


def _selection_vector(g):
    rio = jax.lax.broadcasted_iota(jnp.int32, (_N, _N), 0)
    cio = jax.lax.broadcasted_iota(jnp.int32, (_N, _N), 1)
    eye = rio == cio
    zero = jnp.zeros_like(g)
    sq_col = jnp.sum(jnp.where(eye, g, zero), axis=1, keepdims=True)
    sq_row = jnp.sum(jnp.where(eye, g, zero), axis=0, keepdims=True)
    d2 = sq_col + sq_row - 2.0 * g
    cd = jnp.sqrt(jnp.maximum(d2, 0.0))

    vals = cd
    rowmask = jnp.zeros((_N, _N), jnp.float32)
    s15 = jnp.zeros((_N, 1), jnp.float32)
    big = jnp.float32(3.0e38)
    for _ in range(_K1):
        m = jnp.min(vals, axis=1, keepdims=True)
        s15 = s15 + m
        is_min = vals <= m
        fidx = jnp.min(jnp.where(is_min, cio, _N), axis=1, keepdims=True)
        sel = cio == fidx
        rowmask = jnp.where(sel, 1.0, rowmask)
        vals = jnp.where(sel, big, vals)

    mn = jnp.min(s15)
    rio1 = jax.lax.broadcasted_iota(jnp.int32, (_N, 1), 0)
    istar = jnp.min(jnp.where(s15 <= mn, rio1, _N))

    msel = jnp.where(rio == istar, rowmask, zero)
    return jnp.sum(msel, axis=0, keepdims=True) * (1.0 / _K1)


def _mkrum_kernel(xt_ref, out_ref, gacc_ref, mrow_ref):
    i = pl.program_id(0)

    @pl.when(i < _NB)
    def _phase1():
        blk = xt_ref[...]
        part = jax.lax.dot_general(
            blk, blk, (((1,), (1,)), ((), ())),
            preferred_element_type=jnp.float32,
        )

        @pl.when(i == 0)
        def _():
            gacc_ref[...] = part

        @pl.when(i > 0)
        def _():
            gacc_ref[...] = gacc_ref[...] + part

    @pl.when(i == _NB)
    def _boundary():
        mrow_ref[...] = _selection_vector(gacc_ref[...])

    @pl.when(i >= _NB)
    def _phase2():
        out_ref[...] = jax.lax.dot_general(
            mrow_ref[...], xt_ref[...], (((1,), (0,)), ((), ())),
            preferred_element_type=jnp.float32,
        )


@jax.jit
def kernel(input):
    xt = jnp.reshape(jnp.transpose(input, (0, 2, 1)), (_N, _D))

    def in_map(i):
        return (0, jnp.where(i < _NB, i, i - _NB))

    out = pl.pallas_call(
        _mkrum_kernel,
        grid=(2 * _NB,),
        in_specs=[pl.BlockSpec((_N, _C), in_map)],
        out_specs=pl.BlockSpec((1, _C), lambda i: (0, jnp.maximum(i - _NB, 0))),
        out_shape=jax.ShapeDtypeStruct((1, _D), jnp.float32),
        scratch_shapes=[
            pltpu.VMEM((_N, _N), jnp.float32),
            pltpu.VMEM((1, _N), jnp.float32),
        ],
    )(xt)
    return jnp.reshape(out, (1, _D, 1))


# speedup vs baseline: 8.7081x; 1.4695x over previous
import jax
import jax.numpy as jnp
from jax.experimental import pallas as pl
from jax.experimental.pallas import tpu as pltpu


_D = 262144
_N = 32
_K1 = 15
_C = 16384
_NB = _D // _C


def _selection_vector(g):
    rio = jax.lax.broadcasted_iota(jnp.int32, (_N, _N), 0)
    cio = jax.lax.broadcasted_iota(jnp.int32, (_N, _N), 1)
    eye = rio == cio
    zero = jnp.zeros_like(g)
    sq_col = jnp.sum(jnp.where(eye, g, zero), axis=1, keepdims=True)
    sq_row = jnp.sum(jnp.where(eye, g, zero), axis=0, keepdims=True)
    d2 = sq_col + sq_row - 2.0 * g
    cd = jnp.sqrt(jnp.maximum(d2, 0.0))

    vals = cd
    rowmask = jnp.zeros((_N, _N), jnp.float32)
    s15 = jnp.zeros((_N, 1), jnp.float32)
    big = jnp.float32(3.0e38)
    for _ in range(_K1):
        m = jnp.min(vals, axis=1, keepdims=True)
        s15 = s15 + m
        is_min = vals <= m
        fidx = jnp.min(jnp.where(is_min, cio, _N), axis=1, keepdims=True)
        sel = cio == fidx
        rowmask = jnp.where(sel, 1.0, rowmask)
        vals = jnp.where(sel, big, vals)

    mn = jnp.min(s15)
    rio1 = jax.lax.broadcasted_iota(jnp.int32, (_N, 1), 0)
    istar = jnp.min(jnp.where(s15 <= mn, rio1, _N))

    msel = jnp.where(rio == istar, rowmask, zero)
    return jnp.sum(msel, axis=0, keepdims=True) * (1.0 / _K1)


def _mkrum_kernel(xt_ref, out_ref, gacc_ref, xsave_ref):
    i = pl.program_id(0)

    @pl.when(i < _NB)
    def _phase1():
        blk = xt_ref[...]
        part = jax.lax.dot_general(
            blk, blk, (((1,), (1,)), ((), ())),
            preferred_element_type=jnp.float32,
        )

        @pl.when(i == 0)
        def _():
            gacc_ref[...] = part

        @pl.when(i > 0)
        def _():
            gacc_ref[...] = gacc_ref[...] + part

        xsave_ref[i] = blk

    @pl.when(i == _NB)
    def _finish():
        mrow = _selection_vector(gacc_ref[...])
        for j in range(_NB):
            out_ref[0:1, j * _C:(j + 1) * _C] = jax.lax.dot_general(
                mrow, xsave_ref[j], (((1,), (0,)), ((), ())),
                preferred_element_type=jnp.float32,
            )


@jax.jit
def kernel(input):
    xt = jnp.reshape(jnp.transpose(input, (0, 2, 1)), (_N, _D))

    out = pl.pallas_call(
        _mkrum_kernel,
        grid=(_NB + 1,),
        in_specs=[pl.BlockSpec((_N, _C), lambda i: (0, jnp.minimum(i, _NB - 1)))],
        out_specs=pl.BlockSpec((1, _D), lambda i: (0, 0)),
        out_shape=jax.ShapeDtypeStruct((1, _D), jnp.float32),
        scratch_shapes=[
            pltpu.VMEM((_N, _N), jnp.float32),
            pltpu.VMEM((_NB, _N, _C), jnp.float32),
        ],
    )(xt)
    return jnp.reshape(out, (1, _D, 1))


# speedup vs baseline: 10.5764x; 1.2145x over previous
import jax
import jax.numpy as jnp
from jax.experimental import pallas as pl
from jax.experimental.pallas import tpu as pltpu


_D = 262144
_N = 32
_K1 = 15
_C = 32768
_NB = _D // _C


def _selection_vector(g):
    rio = jax.lax.broadcasted_iota(jnp.int32, (_N, _N), 0)
    cio = jax.lax.broadcasted_iota(jnp.int32, (_N, _N), 1)
    eye = rio == cio
    zero = jnp.zeros_like(g)
    sq_col = jnp.sum(jnp.where(eye, g, zero), axis=1, keepdims=True)
    sq_row = jnp.sum(jnp.where(eye, g, zero), axis=0, keepdims=True)
    d2 = sq_col + sq_row - 2.0 * g
    cd = jnp.sqrt(jnp.maximum(d2, 0.0))

    vals = cd
    rowmask = jnp.zeros((_N, _N), jnp.float32)
    s15 = jnp.zeros((_N, 1), jnp.float32)
    big = jnp.float32(3.0e38)
    for _ in range(_K1):
        m = jnp.min(vals, axis=1, keepdims=True)
        s15 = s15 + m
        is_min = vals <= m
        fidx = jnp.min(jnp.where(is_min, cio, _N), axis=1, keepdims=True)
        sel = cio == fidx
        rowmask = jnp.where(sel, 1.0, rowmask)
        vals = jnp.where(sel, big, vals)

    mn = jnp.min(s15)
    rio1 = jax.lax.broadcasted_iota(jnp.int32, (_N, 1), 0)
    istar = jnp.min(jnp.where(s15 <= mn, rio1, _N))

    msel = jnp.where(rio == istar, rowmask, zero)
    return jnp.sum(msel, axis=0, keepdims=True) * (1.0 / _K1)


def _mkrum_kernel(xt_ref, out_ref, gacc_ref, xsave_ref):
    i = pl.program_id(0)

    @pl.when(i < _NB)
    def _phase1():
        blk = xt_ref[...]
        part = jax.lax.dot_general(
            blk, blk, (((1,), (1,)), ((), ())),
            preferred_element_type=jnp.float32,
        )

        @pl.when(i == 0)
        def _():
            gacc_ref[...] = part

        @pl.when(i > 0)
        def _():
            gacc_ref[...] = gacc_ref[...] + part

        xsave_ref[i] = blk

    @pl.when(i == _NB)
    def _finish():
        mrow = _selection_vector(gacc_ref[...])
        for j in range(_NB):
            out_ref[0:1, j * _C:(j + 1) * _C] = jax.lax.dot_general(
                mrow, xsave_ref[j], (((1,), (0,)), ((), ())),
                preferred_element_type=jnp.float32,
            )


@jax.jit
def kernel(input):
    xt = jnp.reshape(jnp.transpose(input, (0, 2, 1)), (_N, _D))

    out = pl.pallas_call(
        _mkrum_kernel,
        grid=(_NB + 1,),
        in_specs=[pl.BlockSpec((_N, _C), lambda i: (0, jnp.minimum(i, _NB - 1)))],
        out_specs=pl.BlockSpec((1, _D), lambda i: (0, 0)),
        out_shape=jax.ShapeDtypeStruct((1, _D), jnp.float32),
        scratch_shapes=[
            pltpu.VMEM((_N, _N), jnp.float32),
            pltpu.VMEM((_NB, _N, _C), jnp.float32),
        ],
    )(xt)
    return jnp.reshape(out, (1, _D, 1))


# speedup vs baseline: 11.0673x; 1.0464x over previous
import jax
import jax.numpy as jnp
from jax.experimental import pallas as pl
from jax.experimental.pallas import tpu as pltpu


_D = 262144
_N = 32
_K1 = 15
_C = 65536
_NB = _D // _C


def _selection_vector(g):
    rio = jax.lax.broadcasted_iota(jnp.int32, (_N, _N), 0)
    cio = jax.lax.broadcasted_iota(jnp.int32, (_N, _N), 1)
    eye = rio == cio
    zero = jnp.zeros_like(g)
    sq_col = jnp.sum(jnp.where(eye, g, zero), axis=1, keepdims=True)
    sq_row = jnp.sum(jnp.where(eye, g, zero), axis=0, keepdims=True)
    d2 = sq_col + sq_row - 2.0 * g
    cd = jnp.sqrt(jnp.maximum(d2, 0.0))

    vals = cd
    rowmask = jnp.zeros((_N, _N), jnp.float32)
    s15 = jnp.zeros((_N, 1), jnp.float32)
    big = jnp.float32(3.0e38)
    for _ in range(_K1):
        m = jnp.min(vals, axis=1, keepdims=True)
        s15 = s15 + m
        is_min = vals <= m
        fidx = jnp.min(jnp.where(is_min, cio, _N), axis=1, keepdims=True)
        sel = cio == fidx
        rowmask = jnp.where(sel, 1.0, rowmask)
        vals = jnp.where(sel, big, vals)

    mn = jnp.min(s15)
    rio1 = jax.lax.broadcasted_iota(jnp.int32, (_N, 1), 0)
    istar = jnp.min(jnp.where(s15 <= mn, rio1, _N))

    msel = jnp.where(rio == istar, rowmask, zero)
    return jnp.sum(msel, axis=0, keepdims=True) * (1.0 / _K1)


def _mkrum_kernel(xt_ref, out_ref, gacc_ref, xsave_ref):
    i = pl.program_id(0)

    @pl.when(i < _NB)
    def _phase1():
        blk = xt_ref[...]
        part = jax.lax.dot_general(
            blk, blk, (((1,), (1,)), ((), ())),
            preferred_element_type=jnp.float32,
        )

        @pl.when(i == 0)
        def _():
            gacc_ref[...] = part

        @pl.when(i > 0)
        def _():
            gacc_ref[...] = gacc_ref[...] + part

        xsave_ref[i] = blk

    @pl.when(i == _NB)
    def _finish():
        mrow = _selection_vector(gacc_ref[...])
        for j in range(_NB):
            out_ref[0:1, j * _C:(j + 1) * _C] = jax.lax.dot_general(
                mrow, xsave_ref[j], (((1,), (0,)), ((), ())),
                preferred_element_type=jnp.float32,
            )


@jax.jit
def kernel(input):
    xt = jnp.reshape(jnp.transpose(input, (0, 2, 1)), (_N, _D))

    out = pl.pallas_call(
        _mkrum_kernel,
        grid=(_NB + 1,),
        in_specs=[pl.BlockSpec((_N, _C), lambda i: (0, jnp.minimum(i, _NB - 1)))],
        out_specs=pl.BlockSpec((1, _D), lambda i: (0, 0)),
        out_shape=jax.ShapeDtypeStruct((1, _D), jnp.float32),
        scratch_shapes=[
            pltpu.VMEM((_N, _N), jnp.float32),
            pltpu.VMEM((_NB, _N, _C), jnp.float32),
        ],
    )(xt)
    return jnp.reshape(out, (1, _D, 1))
